# Initial kernel scaffold; baseline (speedup 1.0000x reference)
import jax
import jax.numpy as jnp
from jax.experimental import pallas as pl


def kernel(img, salience_map):
    raise NotImplementedError("write your pallas kernel here")


# speedup vs baseline: 2.5044x; 2.5044x over previous
import functools

import jax
import jax.numpy as jnp
from jax.experimental import pallas as pl
from jax.experimental.pallas import tpu as pltpu

_NUM_POINTS = 32
_CROP = 224
_THRESHOLD = 0.15


def _sample_yx(salience_map):
    H, W = salience_map.shape
    prob = salience_map.reshape(-1)
    y_t = max(_CROP // 2, int(_THRESHOLD * H))
    x_t = max(_CROP // 2, int(_THRESHOLD * H))
    border_mask = jnp.zeros((H, W), dtype=salience_map.dtype)
    border_mask = border_mask.at[y_t:H - y_t, x_t:W - x_t].set(1.0)
    border_mask = border_mask.reshape(-1)
    p = prob * border_mask
    p = p / p.sum()
    p = jax.lax.stop_gradient(p)
    skey = jax.random.key(42)
    idx = jax.random.choice(skey, prob.shape[0], shape=(_NUM_POINTS,),
                            replace=True, p=p)
    y = idx // W
    x = idx % W
    return y, x


def _crop_kernel(top_ref, left_ref, img_ref, out_ref):
    i = pl.program_id(0)
    t = top_ref[i]
    l = left_ref[i]
    a0 = t // 8
    dt = t - a0 * 8
    slab = img_ref[:, pl.ds(a0, 29), :, :]
    slab = slab.reshape(3, 232, 512)
    slab = pltpu.roll(slab, -l, axis=2)[:, :, :_CROP]
    out_ref[0] = jax.lax.switch(
        dt, [(lambda d: (lambda: slab[:, d:d + _CROP, :]))(d)
             for d in range(8)])


def kernel(img, salience_map):
    y, x = _sample_yx(salience_map)
    half = _CROP // 2
    top = (y - half).astype(jnp.int32)
    left = (x - half).astype(jnp.int32)
    C, H, W = img.shape

    out = pl.pallas_call(
        _crop_kernel,
        grid=(_NUM_POINTS,),
        in_specs=[
            pl.BlockSpec(memory_space=pltpu.SMEM),
            pl.BlockSpec(memory_space=pltpu.SMEM),
            pl.BlockSpec((C, H // 8, 8, W), lambda i: (0, 0, 0, 0)),
        ],
        out_specs=pl.BlockSpec((1, C, _CROP, _CROP), lambda i: (i, 0, 0, 0)),
        out_shape=jax.ShapeDtypeStruct((_NUM_POINTS, C, _CROP, _CROP),
                                       img.dtype),
    )(top, left, img.reshape(C, H // 8, 8, W))
    return out
